# baseline (speedup 1.0000x reference)
import jax
import jax.numpy as jnp
from jax.experimental import pallas as pl
from jax.experimental.pallas import tpu as pltpu

_NUM_EXPERTS = 64
_TOP_K = 8
_BT = 1024


def _router_block(x_ref, w_ref, wout_ref, iout_ref):
    s = jax.lax.dot_general(
        x_ref[...], w_ref[...],
        dimension_numbers=(((1,), (1,)), ((), ())),
        preferred_element_type=jnp.float32,
    )
    st = s.T
    m = jnp.max(st, axis=0, keepdims=True)
    e = jnp.exp(st - m)
    rscale = 1.0 / jnp.sum(e, axis=0, keepdims=True)

    iota = jax.lax.broadcasted_iota(jnp.int32, e.shape, 0)
    vals = []
    idxs = []
    work = e
    for _ in range(_TOP_K):
        mx = jnp.max(work, axis=0, keepdims=True)
        idx = jnp.min(jnp.where(work == mx, iota, _NUM_EXPERTS),
                      axis=0, keepdims=True)
        vals.append(mx)
        idxs.append(idx)
        work = jnp.where(iota == idx, -1.0, work)
    iout_ref[...] = jnp.concatenate(idxs, axis=0).T
    wout_ref[...] = (jnp.concatenate(vals, axis=0) * rscale).T


@jax.jit
def kernel(x, weight):
    n_tokens, _ = x.shape
    grid = (n_tokens // _BT,)
    wout, iout = pl.pallas_call(
        _router_block,
        grid=grid,
        in_specs=[
            pl.BlockSpec((_BT, x.shape[1]), lambda i: (i, 0)),
            pl.BlockSpec(weight.shape, lambda i: (0, 0)),
        ],
        out_specs=[
            pl.BlockSpec((_BT, _TOP_K), lambda i: (i, 0)),
            pl.BlockSpec((_BT, _TOP_K), lambda i: (i, 0)),
        ],
        out_shape=[
            jax.ShapeDtypeStruct((n_tokens, _TOP_K), jnp.float32),
            jax.ShapeDtypeStruct((n_tokens, _TOP_K), jnp.int32),
        ],
        compiler_params=pltpu.CompilerParams(
            dimension_semantics=("parallel",),
        ),
    )(x, weight)
    return wout, iout
